# baseline (speedup 1.0000x reference)
import jax
import jax.numpy as jnp
from jax.experimental import pallas as pl
from jax.experimental.pallas import tpu as pltpu

_B, _H, _W = 16, 384, 384
_C = 32
_HW = _H * _W
_K_TARGET = _HW - int(round(_HW * 0.1))


def _fused_kernel(vals_ref, out_ref, mask_ref, sem):
    b = pl.program_id(0)

    x = vals_ref[0]
    bits = jax.lax.bitcast_convert_type(x, jnp.int32)
    ikey = jnp.where(bits >= 0, bits, bits ^ jnp.int32(0x7FFFFFFF))

    ones_r = jnp.ones((1, _H), dtype=jnp.float32)

    def body(j, k):
        trial = k + (jnp.int32(1) << (jnp.int32(31) - j))
        pred = (ikey < trial).astype(jnp.float32)
        cnt = jnp.sum(ones_r @ pred, keepdims=True)
        return jnp.where(cnt < jnp.float32(_K_TARGET), trial, k)

    k0 = jnp.full((1, 1), jnp.iinfo(jnp.int32).min, dtype=jnp.int32)
    k = jax.lax.fori_loop(0, 32, body, k0)
    dbits = jnp.where(k >= 0, k, k ^ jnp.int32(0x7FFFFFFF))
    d = jax.lax.bitcast_convert_type(dbits, jnp.float32)

    sel = jax.lax.rem(b, 2)

    @pl.when(b >= 2)
    def _():
        for c in range(_C):
            pltpu.make_async_copy(
                mask_ref.at[sel], out_ref.at[b - 2, c], sem).wait()

    mask_ref[sel] = (x > d).astype(jnp.float32)
    for c in range(_C):
        pltpu.make_async_copy(mask_ref.at[sel], out_ref.at[b, c], sem).start()

    @pl.when(b == _B - 1)
    def _():
        for bb in (_B - 2, _B - 1):
            for c in range(_C):
                pltpu.make_async_copy(
                    mask_ref.at[jax.lax.rem(jnp.int32(bb), 2)],
                    out_ref.at[bb, c], sem).wait()


@jax.jit
def kernel(vals):
    out = pl.pallas_call(
        _fused_kernel,
        grid=(_B,),
        in_specs=[pl.BlockSpec((1, _H, _W), lambda b: (b, 0, 0))],
        out_specs=pl.BlockSpec(memory_space=pl.ANY),
        out_shape=jax.ShapeDtypeStruct((_B, _C, _H, _W), jnp.float32),
        scratch_shapes=[
            pltpu.VMEM((2, _H, _W), jnp.float32),
            pltpu.SemaphoreType.DMA,
        ],
    )(vals)
    return out


# speedup vs baseline: 1.1874x; 1.1874x over previous
import jax
import jax.numpy as jnp
from jax.experimental import pallas as pl
from jax.experimental.pallas import tpu as pltpu

_B, _H, _W = 16, 384, 384
_C = 32
_HW = _H * _W
_K_TARGET = _HW - int(round(_HW * 0.1))


def _fused_kernel(vals_ref, out_ref, mask_ref, sem):
    b = pl.program_id(0)

    x = vals_ref[0]
    bits = jax.lax.bitcast_convert_type(x, jnp.int32)
    ikey = jnp.where(bits >= 0, bits, bits ^ jnp.int32(0x7FFFFFFF))

    ikey4 = ikey.reshape(48, 8, _W)

    def body(j, k):
        trial = k + (jnp.int32(1) << (jnp.int32(31) - j))
        pred = (ikey4 < trial[:, :, None]).astype(jnp.int32)
        t = pred[0:24] + pred[24:48]
        t = t[0:12] + t[12:24]
        t = t[0:6] + t[6:12]
        t = t[0:3] + t[3:6]
        t = t[0:1] + t[1:2] + t[2:3]
        cnt = jnp.sum(t, axis=(1, 2))[:, None]
        return jnp.where(cnt < _K_TARGET, trial, k)

    k0 = jnp.full((1, 1), jnp.iinfo(jnp.int32).min, dtype=jnp.int32)
    k = jax.lax.fori_loop(0, 32, body, k0)
    dbits = jnp.where(k >= 0, k, k ^ jnp.int32(0x7FFFFFFF))
    d = jax.lax.bitcast_convert_type(dbits, jnp.float32)

    sel = jax.lax.rem(b, 2)

    @pl.when(b >= 2)
    def _():
        for c in range(_C):
            pltpu.make_async_copy(
                mask_ref.at[sel], out_ref.at[b - 2, c], sem).wait()

    mask_ref[sel] = (x > d).astype(jnp.float32)
    for c in range(_C):
        pltpu.make_async_copy(mask_ref.at[sel], out_ref.at[b, c], sem).start()

    @pl.when(b == _B - 1)
    def _():
        for bb in (_B - 2, _B - 1):
            for c in range(_C):
                pltpu.make_async_copy(
                    mask_ref.at[jax.lax.rem(jnp.int32(bb), 2)],
                    out_ref.at[bb, c], sem).wait()


@jax.jit
def kernel(vals):
    out = pl.pallas_call(
        _fused_kernel,
        grid=(_B,),
        in_specs=[pl.BlockSpec((1, _H, _W), lambda b: (b, 0, 0))],
        out_specs=pl.BlockSpec(memory_space=pl.ANY),
        out_shape=jax.ShapeDtypeStruct((_B, _C, _H, _W), jnp.float32),
        scratch_shapes=[
            pltpu.VMEM((2, _H, _W), jnp.float32),
            pltpu.SemaphoreType.DMA,
        ],
    )(vals)
    return out


# speedup vs baseline: 1.5532x; 1.3081x over previous
import jax
import jax.numpy as jnp
from jax.experimental import pallas as pl
from jax.experimental.pallas import tpu as pltpu

_B, _H, _W = 16, 384, 384
_C = 32
_G = 2
_NSTEP = _B // _G
_HW = _H * _W
_K_TARGET = _HW - int(round(_HW * 0.1))


def _fused_kernel(vals_ref, out_ref, mask_ref, sem):
    b = pl.program_id(0)

    x = vals_ref[...]
    bits = jax.lax.bitcast_convert_type(x, jnp.int32)
    ikey = jnp.where(bits >= 0, bits, bits ^ jnp.int32(0x7FFFFFFF))
    ikey4 = ikey.reshape(_G, 48, 8, _W)

    def body(j, k):
        trial = k + (jnp.int32(1) << (jnp.int32(31) - j))
        pred = (ikey4 < trial).astype(jnp.int32)
        t = pred[:, 0:24] + pred[:, 24:48]
        t = t[:, 0:12] + t[:, 12:24]
        t = t[:, 0:6] + t[:, 6:12]
        t = t[:, 0:3] + t[:, 3:6]
        t = t[:, 0:1] + t[:, 1:2] + t[:, 2:3]
        cnt = jnp.sum(t, axis=(2, 3), keepdims=True)
        return jnp.where(cnt < _K_TARGET, trial, k)

    k0 = jnp.full((_G, 1, 1, 1), jnp.iinfo(jnp.int32).min, dtype=jnp.int32)
    k = jax.lax.fori_loop(0, 32, body, k0)
    dbits = jnp.where(k >= 0, k, k ^ jnp.int32(0x7FFFFFFF))
    d = jax.lax.bitcast_convert_type(dbits, jnp.float32).reshape(_G, 1, 1)

    sel = jax.lax.rem(b, 2)

    @pl.when(b >= 2)
    def _():
        for g in range(_G):
            for c in range(_C):
                pltpu.make_async_copy(
                    mask_ref.at[sel, g],
                    out_ref.at[(b - 2) * _G + g, c], sem).wait()

    mask_ref[sel] = (x > d).astype(jnp.float32)
    for g in range(_G):
        for c in range(_C):
            pltpu.make_async_copy(
                mask_ref.at[sel, g], out_ref.at[b * _G + g, c], sem).start()

    @pl.when(b == _NSTEP - 1)
    def _():
        for bb in (_NSTEP - 2, _NSTEP - 1):
            for g in range(_G):
                for c in range(_C):
                    pltpu.make_async_copy(
                        mask_ref.at[jax.lax.rem(jnp.int32(bb), 2), g],
                        out_ref.at[bb * _G + g, c], sem).wait()


@jax.jit
def kernel(vals):
    out = pl.pallas_call(
        _fused_kernel,
        grid=(_NSTEP,),
        in_specs=[pl.BlockSpec((_G, _H, _W), lambda b: (b, 0, 0))],
        out_specs=pl.BlockSpec(memory_space=pl.ANY),
        out_shape=jax.ShapeDtypeStruct((_B, _C, _H, _W), jnp.float32),
        scratch_shapes=[
            pltpu.VMEM((2, _G, _H, _W), jnp.float32),
            pltpu.SemaphoreType.DMA,
        ],
    )(vals)
    return out
